# baseline (speedup 1.0000x reference)
import functools

import jax
import jax.numpy as jnp
from jax import lax
from jax.experimental import pallas as pl
from jax.experimental.pallas import tpu as pltpu
from jax.experimental.pallas import tpu_sc as plsc

B = 16384
V = 1000000
D = 64
L = 16
W = 128
VTAIL = V - (V % 128)
K = 2048
NWINMAX = 256
LISTCAP = K + L
RS = 40
SEG0 = 16
CNT0 = SEG0 + NWINMAX + 1


def _sc_info():
    try:
        info = plsc.get_sparse_core_info()
        return info.num_cores, info.num_subcores
    except Exception:
        return 2, 16


def _gather_body(uids_hbm, pos_hbm, neg_hbm, userT_hbm, itemT_hbm,
                 utail_hbm, itail_hbm, rows_hbm, pos_out_hbm, neg_out_hbm,
                 idx_v, rl_v, dl_v, base_v, slab_u, slab_i,
                 tail_u, tail_i, ring_v, dum_v, p2u, p2p, p2n, opos_v, oneg_v, cnt_s,
                 sems, semr,
                 *, nc, nw):
    wid = lax.axis_index("s") * nc + lax.axis_index("c")
    rng = V // nw
    lo = wid * rng
    hi = jnp.minimum(lo + rng, VTAIL)
    blo = lo // W
    bhi = (hi + W - 1) // W
    posb = wid * (B // nw)

    lanes = lax.iota(jnp.int32, L)

    pltpu.sync_copy(uids_hbm, idx_v.at[pl.ds(0, B)])
    pltpu.sync_copy(pos_hbm, idx_v.at[pl.ds(B, B)])
    pltpu.sync_copy(neg_hbm, idx_v.at[pl.ds(2 * B, B)])
    pltpu.sync_copy(utail_hbm, tail_u)
    pltpu.sync_copy(itail_hbm, tail_i)

    for _s in range(8):
        cnt_s[_s] = 0

    pats_a = []
    pats_s = []
    for c in range(D // L):
        dv = lanes + c * L
        pats_a.append(dv // 8)
        pats_s.append(dv % 8)

    def row_out(slot, dest):
        pltpu.async_copy(ring_v.at[slot], rows_hbm.at[pl.ds(dest * D, D)], semr)
        fired = cnt_s[2] + 1
        cnt_s[2] = fired

        @pl.when(fired - cnt_s[3] >= RS)
        def _():
            pltpu.make_async_copy(utail_hbm.at[pl.ds(0, D)], dum_v, semr).wait()
            cnt_s[3] = cnt_s[3] + 1

    def pass_body(carry):
        pass_lo, _total = carry
        cnt_s[0] = 0
        cnt_s[1] = 0
        def scan(ch, _):
            rv = idx_v[pl.ds(ch * L, L)]
            inr = jnp.logical_and(rv >= lo, rv < hi)
            csum = plsc.cumsum(jnp.where(inr, 1, 0))
            ordv = cnt_s[1] + csum - 1
            sel = jnp.logical_and(
                inr, jnp.logical_and(ordv >= pass_lo, ordv < pass_lo + K))
            lp = cnt_s[0]
            plsc.store_compressed(rl_v.at[pl.ds(lp, L)], rv, mask=sel)
            destv = ch * L + lanes
            plsc.store_compressed(dl_v.at[pl.ds(lp, L)], destv, mask=sel)
            cnt_s[0] = lp + plsc.all_reduce_population_count(sel)[0]
            cnt_s[1] = cnt_s[1] + plsc.all_reduce_population_count(inr)[0]
            return 0

        lax.fori_loop(0, 3 * B // L, scan, 0)
        total = cnt_s[1]
        nm = cnt_s[0]
        nchunks = (nm + L - 1) // L

        GW = 31
        ngrp = (bhi - blo + GW - 1) // GW

        def fire_window(b, par):
            woff = pl.multiple_of(b * W, W)
            for a in range(D // 8):
                pltpu.async_copy(
                    userT_hbm.at[pl.ds(a * 8, 8), pl.ds(woff, W)],
                    slab_u.at[par * (D // 8) + a], sems)
                pltpu.async_copy(
                    itemT_hbm.at[pl.ds(a * 8, 8), pl.ds(woff, W)],
                    slab_i.at[par * (D // 8) + a], sems)

        def drain_window(par):
            for a in range(D // 8):
                pltpu.make_async_copy(
                    userT_hbm.at[pl.ds(0, 8), pl.ds(0, W)],
                    slab_u.at[par * (D // 8) + a], sems).wait()
                pltpu.make_async_copy(
                    userT_hbm.at[pl.ds(0, 8), pl.ds(0, W)],
                    slab_i.at[par * (D // 8) + a], sems).wait()

        def group(g, _):
            gwlo = blo + g * GW
            gwhi = jnp.minimum(gwlo + GW, bhi)
            glo = gwlo * W
            ghi = gwhi * W

            cnt_s[5] = 0

            def scan2(q, _):
                rv = rl_v[pl.ds(q * L, L)]
                dv = dl_v[pl.ds(q * L, L)]
                valid = (q * L + lanes) < nm
                sel = jnp.logical_and(
                    valid, jnp.logical_and(rv >= glo, rv < ghi))
                lp = cnt_s[5]
                plsc.store_compressed(rl_v.at[pl.ds(LISTCAP + lp, L)], rv,
                                      mask=sel)
                plsc.store_compressed(dl_v.at[pl.ds(LISTCAP + lp, L)], dv,
                                      mask=sel)
                cnt_s[5] = lp + plsc.all_reduce_population_count(sel)[0]
                return 0

            lax.fori_loop(0, nchunks, scan2, 0)
            nm2 = cnt_s[5]

            @pl.when(nm2 > 0)
            def _():
                fire_window(gwlo, 0)

                def window(b, _):
                    par = lax.rem(b - gwlo, 2)
                    drain_window(par)

                    @pl.when(b + 1 < gwhi)
                    def _():
                        fire_window(b + 1, 1 - par)

                    woff = b * W
                    pbase = par * (D // 8)

                    def entry_chunk(q, _):
                        k0 = LISTCAP + q * L
                        rv = rl_v[pl.ds(k0, L)]
                        dv = dl_v[pl.ds(k0, L)]
                        rem = nm2 - q * L
                        basev = jnp.clip(rv - woff, 0, W - 1)
                        for j in range(L):
                            rj = rv[j]

                            @pl.when(jnp.logical_and(
                                j < rem, jnp.logical_and(
                                    rj >= woff, rj < woff + W)))
                            def _():
                                bscal = basev[j]
                                slot = lax.rem(cnt_s[2], RS)
                                isu_j = dv[j] < B
                                lvec = lanes * 0 + bscal

                                @pl.when(isu_j)
                                def _():
                                    for c in range(D // L):
                                        g2 = plsc.load_gather(
                                            slab_u,
                                            [pbase + pats_a[c], pats_s[c],
                                             lvec])
                                        ring_v[slot, pl.ds(c * L, L)] = g2

                                @pl.when(jnp.logical_not(isu_j))
                                def _():
                                    for c in range(D // L):
                                        g2 = plsc.load_gather(
                                            slab_i,
                                            [pbase + pats_a[c], pats_s[c],
                                             lvec])
                                        ring_v[slot, pl.ds(c * L, L)] = g2

                                row_out(slot, dv[j])
                        return 0

                    lax.fori_loop(0, (nm2 + L - 1) // L, entry_chunk, 0)
                    return 0

                lax.fori_loop(gwlo, gwhi, window, 0)
            return 0

        lax.fori_loop(0, ngrp, group, 0)
        return (pass_lo + K, total)

    lax.while_loop(lambda c: c[0] < c[1], pass_body,
                   (jnp.int32(0), jnp.int32(1)))

    def tail_scan(ch, _):
        for t in range(3):
            off = t * B + posb + ch * L
            rv = idx_v[pl.ds(off, L)]
            basev = (rv - VTAIL) * D
            for j in range(L):
                @pl.when(rv[j] >= VTAIL)
                def _():
                    bscal = basev[j]
                    slot = lax.rem(cnt_s[2], RS)
                    if t == 0:
                        for c in range(D // L):
                            ring_v[slot, pl.ds(c * L, L)] = \
                                tail_u[pl.ds(bscal + c * L, L)]
                    else:
                        for c in range(D // L):
                            ring_v[slot, pl.ds(c * L, L)] = \
                                tail_i[pl.ds(bscal + c * L, L)]
                    row_out(slot, off + j)
        return 0

    lax.fori_loop(0, (B // nw) // L, tail_scan, 0)

    rem_rows = cnt_s[2] - cnt_s[3]

    def fin(i, _):
        @pl.when(i < rem_rows)
        def _():
            pltpu.make_async_copy(utail_hbm.at[pl.ds(0, D)], dum_v, semr).wait()
        return 0

    lax.fori_loop(0, RS, fin, 0)

    plsc.subcore_barrier()

    bpw = B // nw
    base = wid * bpw
    PC = 64
    masks = [lanes == j for j in range(L)]

    def p2chunk(k, _):
        pbase = base + k * PC
        c0 = pltpu.async_copy(
            rows_hbm.at[pl.ds(pbase * D, PC * D)], p2u, sems)
        c1 = pltpu.async_copy(
            rows_hbm.at[pl.ds((B + pbase) * D, PC * D)], p2p, sems)
        c2 = pltpu.async_copy(
            rows_hbm.at[pl.ds((2 * B + pbase) * D, PC * D)], p2n, sems)
        c0.wait()
        c1.wait()
        c2.wait()

        def group(g, _):
            vp = jnp.zeros((L,), jnp.float32)
            vn = jnp.zeros((L,), jnp.float32)
            for j in range(L):
                roff = (g * L + j) * D
                ap = jnp.zeros((L,), jnp.float32)
                an = jnp.zeros((L,), jnp.float32)
                for c in range(D // L):
                    u = p2u[pl.ds(roff + c * L, L)]
                    ap = ap + u * p2p[pl.ds(roff + c * L, L)]
                    an = an + u * p2n[pl.ds(roff + c * L, L)]
                vp = jnp.where(masks[j], jnp.sum(ap), vp)
                vn = jnp.where(masks[j], jnp.sum(an), vn)
            opos_v[pl.ds(k * PC + g * L, L)] = vp
            oneg_v[pl.ds(k * PC + g * L, L)] = vn
            return 0

        lax.fori_loop(0, PC // L, group, 0)
        return 0

    lax.fori_loop(0, bpw // PC, p2chunk, 0)

    pltpu.sync_copy(opos_v, pos_out_hbm.at[pl.ds(base, bpw)])
    pltpu.sync_copy(oneg_v, neg_out_hbm.at[pl.ds(base, bpw)])


def _dot_body(rows_hbm, pos_out_hbm, neg_out_hbm,
              u_v, p_v, n_v, opos_v, oneg_v, sem,
              *, nc, nw):
    wid = lax.axis_index("s") * nc + lax.axis_index("c")
    bpw = B // nw
    base = wid * bpw

    cu = pltpu.async_copy(rows_hbm.at[pl.ds(base * D, bpw * D)], u_v, sem)
    cp = pltpu.async_copy(rows_hbm.at[pl.ds((B + base) * D, bpw * D)], p_v, sem)
    cn = pltpu.async_copy(rows_hbm.at[pl.ds((2 * B + base) * D, bpw * D)],
                          n_v, sem)
    cu.wait()
    cp.wait()
    cn.wait()

    lane = lax.iota(jnp.int32, L)
    masks = [lane == j for j in range(L)]

    def group(g, _):
        vp = jnp.zeros((L,), jnp.float32)
        vn = jnp.zeros((L,), jnp.float32)
        for j in range(L):
            roff = (g * L + j) * D
            ap = jnp.zeros((L,), jnp.float32)
            an = jnp.zeros((L,), jnp.float32)
            for c in range(D // L):
                u = u_v[pl.ds(roff + c * L, L)]
                ap = ap + u * p_v[pl.ds(roff + c * L, L)]
                an = an + u * n_v[pl.ds(roff + c * L, L)]
            vp = jnp.where(masks[j], jnp.sum(ap), vp)
            vn = jnp.where(masks[j], jnp.sum(an), vn)
        opos_v[pl.ds(g * L, L)] = vp
        oneg_v[pl.ds(g * L, L)] = vn
        return 0

    lax.fori_loop(0, bpw // L, group, 0)

    pltpu.sync_copy(opos_v, pos_out_hbm.at[pl.ds(base, bpw)])
    pltpu.sync_copy(oneg_v, neg_out_hbm.at[pl.ds(base, bpw)])


def kernel(uids, pos_iids, neg_iids, embed_user, embed_item):
    nc, ns = _sc_info()
    nw = nc * ns
    mesh = plsc.VectorSubcoreMesh(core_axis_name="c", subcore_axis_name="s")

    fused = pl.kernel(
        functools.partial(_gather_body, nc=nc, nw=nw),
        out_type=(
            jax.ShapeDtypeStruct((3 * B * D,), jnp.float32),
            jax.ShapeDtypeStruct((B,), jnp.float32),
            jax.ShapeDtypeStruct((B,), jnp.float32),
        ),
        mesh=mesh,
        scratch_types=[
            pltpu.VMEM((3 * B,), jnp.int32),
            pltpu.VMEM((2 * LISTCAP,), jnp.int32),
            pltpu.VMEM((2 * LISTCAP,), jnp.int32),
            pltpu.VMEM((NWINMAX,), jnp.int32),
            pltpu.VMEM((2 * (D // 8), 8, W), jnp.float32),
            pltpu.VMEM((2 * (D // 8), 8, W), jnp.float32),
            pltpu.VMEM((64 * D,), jnp.float32),
            pltpu.VMEM((64 * D,), jnp.float32),
            pltpu.VMEM((RS, D), jnp.float32),
            pltpu.VMEM((D,), jnp.float32),
            pltpu.VMEM((64 * D,), jnp.float32),
            pltpu.VMEM((64 * D,), jnp.float32),
            pltpu.VMEM((64 * D,), jnp.float32),
            pltpu.VMEM((B // nw,), jnp.float32),
            pltpu.VMEM((B // nw,), jnp.float32),
            pltpu.SMEM((CNT0 + NWINMAX + 8,), jnp.int32),
            pltpu.SemaphoreType.DMA,
            pltpu.SemaphoreType.DMA,
        ],
        compiler_params=pltpu.CompilerParams(needs_layout_passes=False),
    )

    utail = embed_user[VTAIL:].reshape(-1)
    itail = embed_item[VTAIL:].reshape(-1)
    _rows, pos_preds, neg_preds = fused(
        uids, pos_iids, neg_iids, embed_user.T, embed_item.T, utail, itail)
    return (pos_preds, neg_preds)


# speedup vs baseline: 1.0556x; 1.0556x over previous
import functools

import jax
import jax.numpy as jnp
from jax import lax
from jax.experimental import pallas as pl
from jax.experimental.pallas import tpu as pltpu
from jax.experimental.pallas import tpu_sc as plsc

B = 16384
V = 1000000
D = 64
L = 16
W = 128
VTAIL = V - (V % 128)
K = 2048
NWINMAX = 256
LISTCAP = K + L
RS = 40
SEG0 = 16
CNT0 = SEG0 + NWINMAX + 1


def _sc_info():
    try:
        info = plsc.get_sparse_core_info()
        return info.num_cores, info.num_subcores
    except Exception:
        return 2, 16


def _gather_body(uids_hbm, pos_hbm, neg_hbm, userT_hbm, itemT_hbm,
                 utail_hbm, itail_hbm, rows_hbm, pos_out_hbm, neg_out_hbm,
                 idx_v, rl_v, dl_v, base_v, slab_u, slab_i,
                 tail_u, tail_i, ring_v, dum_v, p2u, p2p, p2n, opos_v, oneg_v, cnt_s,
                 sems, semr,
                 *, nc, nw):
    wid = lax.axis_index("s") * nc + lax.axis_index("c")
    rng = V // nw
    lo = wid * rng
    hi = jnp.minimum(lo + rng, VTAIL)
    blo = lo // W
    bhi = (hi + W - 1) // W
    posb = wid * (B // nw)

    lanes = lax.iota(jnp.int32, L)

    pltpu.sync_copy(uids_hbm, idx_v.at[pl.ds(0, B)])
    pltpu.sync_copy(pos_hbm, idx_v.at[pl.ds(B, B)])
    pltpu.sync_copy(neg_hbm, idx_v.at[pl.ds(2 * B, B)])
    pltpu.sync_copy(utail_hbm, tail_u)
    pltpu.sync_copy(itail_hbm, tail_i)

    for _s in range(8):
        cnt_s[_s] = 0

    pats_d = [lanes + c * L for c in range(D // L)]

    def row_out(slot, dest):
        pltpu.async_copy(ring_v.at[slot], rows_hbm.at[pl.ds(dest * D, D)], semr)
        fired = cnt_s[2] + 1
        cnt_s[2] = fired

        @pl.when(fired - cnt_s[3] >= RS)
        def _():
            pltpu.make_async_copy(utail_hbm.at[pl.ds(0, D)], dum_v, semr).wait()
            cnt_s[3] = cnt_s[3] + 1

    def pass_body(carry):
        pass_lo, _total = carry
        cnt_s[0] = 0
        cnt_s[1] = 0
        def scan(ch, _):
            rv = idx_v[pl.ds(ch * L, L)]
            inr = jnp.logical_and(rv >= lo, rv < hi)
            csum = plsc.cumsum(jnp.where(inr, 1, 0))
            ordv = cnt_s[1] + csum - 1
            sel = jnp.logical_and(
                inr, jnp.logical_and(ordv >= pass_lo, ordv < pass_lo + K))
            lp = cnt_s[0]
            plsc.store_compressed(rl_v.at[pl.ds(lp, L)], rv, mask=sel)
            destv = ch * L + lanes
            plsc.store_compressed(dl_v.at[pl.ds(lp, L)], destv, mask=sel)
            cnt_s[0] = lp + plsc.all_reduce_population_count(sel)[0]
            cnt_s[1] = cnt_s[1] + plsc.all_reduce_population_count(inr)[0]
            return 0

        lax.fori_loop(0, 3 * B // L, scan, 0)
        total = cnt_s[1]
        nm = cnt_s[0]
        nchunks = (nm + L - 1) // L

        GW = 31
        ngrp = (bhi - blo + GW - 1) // GW

        def fire_window(b, par):
            woff = pl.multiple_of(b * W, W)
            pltpu.async_copy(
                userT_hbm.at[pl.ds(0, D), pl.ds(woff, W)], slab_u.at[par], sems)
            pltpu.async_copy(
                itemT_hbm.at[pl.ds(0, D), pl.ds(woff, W)], slab_i.at[par], sems)

        def drain_window(par):
            pltpu.make_async_copy(
                userT_hbm.at[pl.ds(0, D), pl.ds(0, W)],
                slab_u.at[par], sems).wait()
            pltpu.make_async_copy(
                userT_hbm.at[pl.ds(0, D), pl.ds(0, W)],
                slab_i.at[par], sems).wait()

        def group(g, _):
            gwlo = blo + g * GW
            gwhi = jnp.minimum(gwlo + GW, bhi)
            glo = gwlo * W
            ghi = gwhi * W

            cnt_s[5] = 0

            def scan2(q, _):
                rv = rl_v[pl.ds(q * L, L)]
                dv = dl_v[pl.ds(q * L, L)]
                valid = (q * L + lanes) < nm
                sel = jnp.logical_and(
                    valid, jnp.logical_and(rv >= glo, rv < ghi))
                lp = cnt_s[5]
                plsc.store_compressed(rl_v.at[pl.ds(LISTCAP + lp, L)], rv,
                                      mask=sel)
                plsc.store_compressed(dl_v.at[pl.ds(LISTCAP + lp, L)], dv,
                                      mask=sel)
                cnt_s[5] = lp + plsc.all_reduce_population_count(sel)[0]
                return 0

            lax.fori_loop(0, nchunks, scan2, 0)
            nm2 = cnt_s[5]

            @pl.when(nm2 > 0)
            def _():
                fire_window(gwlo, 0)

                def window(b, _):
                    par = lax.rem(b - gwlo, 2)
                    drain_window(par)

                    @pl.when(b + 1 < gwhi)
                    def _():
                        fire_window(b + 1, 1 - par)

                    woff = b * W

                    def entry_chunk(q, _):
                        k0 = LISTCAP + q * L
                        rv = rl_v[pl.ds(k0, L)]
                        dv = dl_v[pl.ds(k0, L)]
                        rem = nm2 - q * L
                        basev = jnp.clip(rv - woff, 0, W - 1)
                        for j in range(L):
                            rj = rv[j]

                            @pl.when(jnp.logical_and(
                                j < rem, jnp.logical_and(
                                    rj >= woff, rj < woff + W)))
                            def _():
                                bscal = basev[j]
                                slot = lax.rem(cnt_s[2], RS)
                                isu_j = dv[j] < B
                                lvec = lanes * 0 + bscal

                                pvec = lanes * 0 + par

                                @pl.when(isu_j)
                                def _():
                                    for c in range(D // L):
                                        g2 = plsc.load_gather(
                                            slab_u, [pvec, pats_d[c], lvec])
                                        ring_v[slot, pl.ds(c * L, L)] = g2

                                @pl.when(jnp.logical_not(isu_j))
                                def _():
                                    for c in range(D // L):
                                        g2 = plsc.load_gather(
                                            slab_i, [pvec, pats_d[c], lvec])
                                        ring_v[slot, pl.ds(c * L, L)] = g2

                                row_out(slot, dv[j])
                        return 0

                    lax.fori_loop(0, (nm2 + L - 1) // L, entry_chunk, 0)
                    return 0

                lax.fori_loop(gwlo, gwhi, window, 0)
            return 0

        lax.fori_loop(0, ngrp, group, 0)
        return (pass_lo + K, total)

    lax.while_loop(lambda c: c[0] < c[1], pass_body,
                   (jnp.int32(0), jnp.int32(1)))

    def tail_scan(ch, _):
        for t in range(3):
            off = t * B + posb + ch * L
            rv = idx_v[pl.ds(off, L)]
            basev = (rv - VTAIL) * D
            for j in range(L):
                @pl.when(rv[j] >= VTAIL)
                def _():
                    bscal = basev[j]
                    slot = lax.rem(cnt_s[2], RS)
                    if t == 0:
                        for c in range(D // L):
                            ring_v[slot, pl.ds(c * L, L)] = \
                                tail_u[pl.ds(bscal + c * L, L)]
                    else:
                        for c in range(D // L):
                            ring_v[slot, pl.ds(c * L, L)] = \
                                tail_i[pl.ds(bscal + c * L, L)]
                    row_out(slot, off + j)
        return 0

    lax.fori_loop(0, (B // nw) // L, tail_scan, 0)

    rem_rows = cnt_s[2] - cnt_s[3]

    def fin(i, _):
        @pl.when(i < rem_rows)
        def _():
            pltpu.make_async_copy(utail_hbm.at[pl.ds(0, D)], dum_v, semr).wait()
        return 0

    lax.fori_loop(0, RS, fin, 0)

    plsc.subcore_barrier()

    bpw = B // nw
    base = wid * bpw
    PC = 64
    masks = [lanes == j for j in range(L)]

    def p2chunk(k, _):
        pbase = base + k * PC
        c0 = pltpu.async_copy(
            rows_hbm.at[pl.ds(pbase * D, PC * D)], p2u, sems)
        c1 = pltpu.async_copy(
            rows_hbm.at[pl.ds((B + pbase) * D, PC * D)], p2p, sems)
        c2 = pltpu.async_copy(
            rows_hbm.at[pl.ds((2 * B + pbase) * D, PC * D)], p2n, sems)
        c0.wait()
        c1.wait()
        c2.wait()

        def group(g, _):
            vp = jnp.zeros((L,), jnp.float32)
            vn = jnp.zeros((L,), jnp.float32)
            for j in range(L):
                roff = (g * L + j) * D
                ap = jnp.zeros((L,), jnp.float32)
                an = jnp.zeros((L,), jnp.float32)
                for c in range(D // L):
                    u = p2u[pl.ds(roff + c * L, L)]
                    ap = ap + u * p2p[pl.ds(roff + c * L, L)]
                    an = an + u * p2n[pl.ds(roff + c * L, L)]
                vp = jnp.where(masks[j], jnp.sum(ap), vp)
                vn = jnp.where(masks[j], jnp.sum(an), vn)
            opos_v[pl.ds(k * PC + g * L, L)] = vp
            oneg_v[pl.ds(k * PC + g * L, L)] = vn
            return 0

        lax.fori_loop(0, PC // L, group, 0)
        return 0

    lax.fori_loop(0, bpw // PC, p2chunk, 0)

    pltpu.sync_copy(opos_v, pos_out_hbm.at[pl.ds(base, bpw)])
    pltpu.sync_copy(oneg_v, neg_out_hbm.at[pl.ds(base, bpw)])


def _dot_body(rows_hbm, pos_out_hbm, neg_out_hbm,
              u_v, p_v, n_v, opos_v, oneg_v, sem,
              *, nc, nw):
    wid = lax.axis_index("s") * nc + lax.axis_index("c")
    bpw = B // nw
    base = wid * bpw

    cu = pltpu.async_copy(rows_hbm.at[pl.ds(base * D, bpw * D)], u_v, sem)
    cp = pltpu.async_copy(rows_hbm.at[pl.ds((B + base) * D, bpw * D)], p_v, sem)
    cn = pltpu.async_copy(rows_hbm.at[pl.ds((2 * B + base) * D, bpw * D)],
                          n_v, sem)
    cu.wait()
    cp.wait()
    cn.wait()

    lane = lax.iota(jnp.int32, L)
    masks = [lane == j for j in range(L)]

    def group(g, _):
        vp = jnp.zeros((L,), jnp.float32)
        vn = jnp.zeros((L,), jnp.float32)
        for j in range(L):
            roff = (g * L + j) * D
            ap = jnp.zeros((L,), jnp.float32)
            an = jnp.zeros((L,), jnp.float32)
            for c in range(D // L):
                u = u_v[pl.ds(roff + c * L, L)]
                ap = ap + u * p_v[pl.ds(roff + c * L, L)]
                an = an + u * n_v[pl.ds(roff + c * L, L)]
            vp = jnp.where(masks[j], jnp.sum(ap), vp)
            vn = jnp.where(masks[j], jnp.sum(an), vn)
        opos_v[pl.ds(g * L, L)] = vp
        oneg_v[pl.ds(g * L, L)] = vn
        return 0

    lax.fori_loop(0, bpw // L, group, 0)

    pltpu.sync_copy(opos_v, pos_out_hbm.at[pl.ds(base, bpw)])
    pltpu.sync_copy(oneg_v, neg_out_hbm.at[pl.ds(base, bpw)])


def kernel(uids, pos_iids, neg_iids, embed_user, embed_item):
    nc, ns = _sc_info()
    nw = nc * ns
    mesh = plsc.VectorSubcoreMesh(core_axis_name="c", subcore_axis_name="s")

    fused = pl.kernel(
        functools.partial(_gather_body, nc=nc, nw=nw),
        out_type=(
            jax.ShapeDtypeStruct((3 * B * D,), jnp.float32),
            jax.ShapeDtypeStruct((B,), jnp.float32),
            jax.ShapeDtypeStruct((B,), jnp.float32),
        ),
        mesh=mesh,
        scratch_types=[
            pltpu.VMEM((3 * B,), jnp.int32),
            pltpu.VMEM((2 * LISTCAP,), jnp.int32),
            pltpu.VMEM((2 * LISTCAP,), jnp.int32),
            pltpu.VMEM((NWINMAX,), jnp.int32),
            pltpu.VMEM((2, D, W), jnp.float32),
            pltpu.VMEM((2, D, W), jnp.float32),
            pltpu.VMEM((64 * D,), jnp.float32),
            pltpu.VMEM((64 * D,), jnp.float32),
            pltpu.VMEM((RS, D), jnp.float32),
            pltpu.VMEM((D,), jnp.float32),
            pltpu.VMEM((64 * D,), jnp.float32),
            pltpu.VMEM((64 * D,), jnp.float32),
            pltpu.VMEM((64 * D,), jnp.float32),
            pltpu.VMEM((B // nw,), jnp.float32),
            pltpu.VMEM((B // nw,), jnp.float32),
            pltpu.SMEM((CNT0 + NWINMAX + 8,), jnp.int32),
            pltpu.SemaphoreType.DMA,
            pltpu.SemaphoreType.DMA,
        ],
        compiler_params=pltpu.CompilerParams(needs_layout_passes=False),
    )

    utail = embed_user[VTAIL:].reshape(-1)
    itail = embed_item[VTAIL:].reshape(-1)
    _rows, pos_preds, neg_preds = fused(
        uids, pos_iids, neg_iids, embed_user.T, embed_item.T, utail, itail)
    return (pos_preds, neg_preds)


# speedup vs baseline: 3.0537x; 2.8928x over previous
import functools

import jax
import jax.numpy as jnp
from jax import lax
from jax.experimental import pallas as pl
from jax.experimental.pallas import tpu as pltpu
from jax.experimental.pallas import tpu_sc as plsc

B = 16384
D = 64
L = 16
CH = 256


def _sc_info():
    try:
        info = plsc.get_sparse_core_info()
        return info.num_cores, info.num_subcores
    except Exception:
        return 2, 16


def _body(uids_hbm, pos_hbm, neg_hbm, user_hbm, item_hbm,
          pos_out_hbm, neg_out_hbm,
          iu_v, ip_v, in_v, u_v, p_v, n_v, opos_v, oneg_v, sem,
          *, nc, bpw):
    wid = lax.axis_index("s") * nc + lax.axis_index("c")
    base = wid * bpw

    pltpu.sync_copy(uids_hbm.at[pl.ds(base, bpw)], iu_v)
    pltpu.sync_copy(pos_hbm.at[pl.ds(base, bpw)], ip_v)
    pltpu.sync_copy(neg_hbm.at[pl.ds(base, bpw)], in_v)

    lane = lax.iota(jnp.int32, L)
    masks = [lane == j for j in range(L)]

    for ch in range(bpw // CH):
        rbase = ch * CH

        def fire(g, _):
            idu = iu_v[pl.ds(rbase + g * L, L)]
            idp = ip_v[pl.ds(rbase + g * L, L)]
            idn = in_v[pl.ds(rbase + g * L, L)]
            for j in range(L):
                r = g * L + j
                pltpu.async_copy(user_hbm.at[idu[j]], u_v.at[r], sem)
                pltpu.async_copy(item_hbm.at[idp[j]], p_v.at[r], sem)
                pltpu.async_copy(item_hbm.at[idn[j]], n_v.at[r], sem)
            return 0

        lax.fori_loop(0, CH // L, fire, 0)

        def drain(g, _):
            for j in range(L):
                r = g * L + j
                pltpu.make_async_copy(user_hbm.at[0], u_v.at[r], sem).wait()
                pltpu.make_async_copy(item_hbm.at[0], p_v.at[r], sem).wait()
                pltpu.make_async_copy(item_hbm.at[0], n_v.at[r], sem).wait()
            return 0

        lax.fori_loop(0, CH // L, drain, 0)

        def group(g, _):
            vp = jnp.zeros((L,), jnp.float32)
            vn = jnp.zeros((L,), jnp.float32)
            for j in range(L):
                r = g * L + j
                ap = jnp.zeros((L,), jnp.float32)
                an = jnp.zeros((L,), jnp.float32)
                for c in range(D // L):
                    u = u_v[r, pl.ds(c * L, L)]
                    ap = ap + u * p_v[r, pl.ds(c * L, L)]
                    an = an + u * n_v[r, pl.ds(c * L, L)]
                vp = jnp.where(masks[j], jnp.sum(ap), vp)
                vn = jnp.where(masks[j], jnp.sum(an), vn)
            opos_v[pl.ds(rbase + g * L, L)] = vp
            oneg_v[pl.ds(rbase + g * L, L)] = vn
            return 0

        lax.fori_loop(0, CH // L, group, 0)

    pltpu.sync_copy(opos_v, pos_out_hbm.at[pl.ds(base, bpw)])
    pltpu.sync_copy(oneg_v, neg_out_hbm.at[pl.ds(base, bpw)])


def kernel(uids, pos_iids, neg_iids, embed_user, embed_item):
    nc, ns = _sc_info()
    nw = nc * ns
    bpw = B // nw
    mesh = plsc.VectorSubcoreMesh(core_axis_name="c", subcore_axis_name="s")
    k = pl.kernel(
        functools.partial(_body, nc=nc, bpw=bpw),
        out_type=(
            jax.ShapeDtypeStruct((B,), jnp.float32),
            jax.ShapeDtypeStruct((B,), jnp.float32),
        ),
        mesh=mesh,
        scratch_types=[
            pltpu.VMEM((bpw,), jnp.int32),
            pltpu.VMEM((bpw,), jnp.int32),
            pltpu.VMEM((bpw,), jnp.int32),
            pltpu.VMEM((CH, D), jnp.float32),
            pltpu.VMEM((CH, D), jnp.float32),
            pltpu.VMEM((CH, D), jnp.float32),
            pltpu.VMEM((bpw,), jnp.float32),
            pltpu.VMEM((bpw,), jnp.float32),
            pltpu.SemaphoreType.DMA,
        ],
        compiler_params=pltpu.CompilerParams(needs_layout_passes=False),
    )
    return k(uids, pos_iids, neg_iids, embed_user, embed_item)
